# Initial kernel scaffold; baseline (speedup 1.0000x reference)
import jax
import jax.numpy as jnp
from jax.experimental import pallas as pl


def kernel(edge_index, pseudo, features, weight, bias):
    raise NotImplementedError("write your pallas kernel here")


# speedup vs baseline: 13.6588x; 13.6588x over previous
import jax
import jax.numpy as jnp
from jax import lax
from jax.experimental import pallas as pl
from jax.experimental.pallas import tpu as pltpu
from jax.experimental.pallas import tpu_sc as plsc

NC = 2
NS = 16
LANES = 16

KK = 4
CE = 40


def _xw_body(x_ref, w_ref, o_ref):
    o_ref[...] = jnp.dot(x_ref[...], w_ref[...],
                         preferred_element_type=jnp.float32)


def _compute_xw(features, w2, n_blk):
    n, d_in = features.shape
    d_cols = w2.shape[1]
    return pl.pallas_call(
        _xw_body,
        grid=(n // n_blk,),
        in_specs=[
            pl.BlockSpec((n_blk, d_in), lambda i: (i, 0)),
            pl.BlockSpec((d_in, d_cols), lambda i: (0, 0)),
        ],
        out_specs=pl.BlockSpec((n_blk, d_cols), lambda i: (i, 0)),
        out_shape=jax.ShapeDtypeStruct((n, d_cols), jnp.float32),
    )(features, w2)


def _sc_body(src_hbm, dst_hbm, p0_hbm, p1_hbm, xw_hbm, outp_hbm, degp_hbm,
             src_v, dst_v, ps0_v, ps1_v, bas_v, rows_v, msg_v, ones_v,
             degcol_v, acc_sh, deg_sh):
    c = lax.axis_index("c")
    s = lax.axis_index("s")
    wid = c * NS + s

    n_nodes = acc_sh.shape[0]
    d_out = acc_sh.shape[1]
    rpt = (n_nodes // (NS * 8)) * 8
    tail = n_nodes - NS * rpt
    e_total = src_hbm.shape[0]
    e_per_w = e_total // (NC * NS)
    n_chunks = e_per_w // CE

    zero16 = jnp.zeros((LANES,), jnp.float32)
    one16 = jnp.ones((LANES,), jnp.float32)

    @pl.loop(0, CE)
    def _zz(i):
        for v in range(d_out // LANES):
            msg_v[i, pl.ds(v * LANES, LANES)] = zero16

    for j in range(rpt // CE):
        pltpu.sync_copy(msg_v, acc_sh.at[pl.ds(s * rpt + j * CE, CE)])
    zt = rpt - (rpt // CE) * CE
    if zt:
        pltpu.sync_copy(msg_v.at[pl.ds(0, zt)],
                        acc_sh.at[pl.ds(s * rpt + (rpt // CE) * CE, zt)])

    @pl.loop(0, rpt // LANES)
    def _zd(i):
        degcol_v[pl.ds(i * LANES, LANES)] = zero16

    pltpu.sync_copy(degcol_v, deg_sh.at[pl.ds(s * rpt, rpt)])

    @pl.when(s == 0)
    def _ztail():
        pltpu.sync_copy(msg_v.at[pl.ds(0, tail)],
                        acc_sh.at[pl.ds(NS * rpt, tail)])
        pltpu.sync_copy(degcol_v.at[pl.ds(0, tail)],
                        deg_sh.at[pl.ds(NS * rpt, tail)])

    offs = list(range(0, CE - LANES + 1, LANES))
    if CE % LANES:
        offs.append(CE - LANES)

    for o in offs:
        ones_v[pl.ds(o, LANES)] = one16

    plsc.subcore_barrier()

    base_w = wid * e_per_w

    @pl.loop(0, n_chunks)
    def _chunk(ci):
        base = base_w + ci * CE
        pltpu.sync_copy(src_hbm.at[pl.ds(base, CE)], src_v)
        pltpu.sync_copy(dst_hbm.at[pl.ds(base, CE)], dst_v)
        pltpu.sync_copy(p0_hbm.at[pl.ds(base, CE)], ps0_v)
        pltpu.sync_copy(p1_hbm.at[pl.ds(base, CE)], ps1_v)

        for o in offs:
            p0 = ps0_v[pl.ds(o, LANES)]
            p1 = ps1_v[pl.ds(o, LANES)]
            q0 = 1.0 - p0
            q1 = 1.0 - p1
            bas_v[0, pl.ds(o, LANES)] = q0 * q1
            bas_v[1, pl.ds(o, LANES)] = q0 * p1
            bas_v[2, pl.ds(o, LANES)] = p0 * q1
            bas_v[3, pl.ds(o, LANES)] = p0 * p1

        pltpu.sync_copy(xw_hbm.at[src_v], rows_v)

        @pl.loop(0, CE)
        def _edge(e):
            b = [bas_v[k, pl.ds(e, LANES)][0] for k in range(KK)]
            for v in range(d_out // LANES):
                acc = b[0] * rows_v[e, pl.ds(v * LANES, LANES)]
                for k in range(1, KK):
                    acc = acc + b[k] * rows_v[e, pl.ds(k * d_out + v * LANES,
                                                       LANES)]
                msg_v[e, pl.ds(v * LANES, LANES)] = acc

        pltpu.sync_copy(msg_v, acc_sh.at[dst_v], add=True)
        pltpu.sync_copy(ones_v, deg_sh.at[dst_v], add=True)

    plsc.subcore_barrier()

    for j in range(rpt // CE):
        pltpu.sync_copy(acc_sh.at[pl.ds(s * rpt + j * CE, CE)], msg_v)
        pltpu.sync_copy(msg_v, outp_hbm.at[c, pl.ds(s * rpt + j * CE, CE)])
    if zt:
        pltpu.sync_copy(acc_sh.at[pl.ds(s * rpt + (rpt // CE) * CE, zt)],
                        msg_v.at[pl.ds(0, zt)])
        pltpu.sync_copy(msg_v.at[pl.ds(0, zt)],
                        outp_hbm.at[c, pl.ds(s * rpt + (rpt // CE) * CE,
                                             zt)])

    pltpu.sync_copy(deg_sh.at[pl.ds(s * rpt, rpt)], degcol_v)
    pltpu.sync_copy(degcol_v,
                    degp_hbm.at[pl.ds(c * n_nodes + s * rpt, rpt)])

    @pl.when(s == 0)
    def _dtail():
        pltpu.sync_copy(acc_sh.at[pl.ds(NS * rpt, tail)],
                        msg_v.at[pl.ds(0, tail)])
        pltpu.sync_copy(msg_v.at[pl.ds(0, tail)],
                        outp_hbm.at[c, pl.ds(NS * rpt, tail)])
        pltpu.sync_copy(deg_sh.at[pl.ds(NS * rpt, tail)],
                        degcol_v.at[pl.ds(0, tail)])
        pltpu.sync_copy(degcol_v.at[pl.ds(0, tail)],
                        degp_hbm.at[pl.ds(c * n_nodes + NS * rpt, tail)])


def _sc_scatter(src, dst, p0, p1, xw, n_nodes, d_out):
    mesh = plsc.VectorSubcoreMesh(core_axis_name="c", subcore_axis_name="s",
                                  num_cores=NC, num_subcores=NS)
    rpt = (n_nodes // (NS * 8)) * 8
    fn = pl.kernel(
        _sc_body,
        out_type=[
            jax.ShapeDtypeStruct((NC, n_nodes, d_out), jnp.float32),
            jax.ShapeDtypeStruct((NC * n_nodes,), jnp.float32),
        ],
        mesh=mesh,
        compiler_params=pltpu.CompilerParams(needs_layout_passes=False),
        scratch_types=[
            pltpu.VMEM((CE,), jnp.int32),
            pltpu.VMEM((CE,), jnp.int32),
            pltpu.VMEM((CE,), jnp.float32),
            pltpu.VMEM((CE,), jnp.float32),
            pltpu.VMEM((KK, CE + LANES), jnp.float32),
            pltpu.VMEM((CE, KK * d_out), jnp.float32),
            pltpu.VMEM((CE, d_out), jnp.float32),
            pltpu.VMEM((CE,), jnp.float32),
            pltpu.VMEM((rpt,), jnp.float32),
            pltpu.MemorySpace.VMEM_SHARED((n_nodes, d_out), jnp.float32),
            pltpu.MemorySpace.VMEM_SHARED((n_nodes,), jnp.float32),
        ],
    )
    return fn(src, dst, p0, p1, xw)


def _fin_body(op_ref, da_ref, db_ref, b_ref, o_ref):
    deg = jnp.clip(da_ref[...] + db_ref[...], 1.0, None)
    o_ref[...] = (op_ref[0] + op_ref[1]) / deg + b_ref[...][None, :]


def _finalize(outp, deg_a, deg_b, bias, n_blk):
    _, n, d_out = outp.shape
    return pl.pallas_call(
        _fin_body,
        grid=(n // n_blk,),
        in_specs=[
            pl.BlockSpec((NC, n_blk, d_out), lambda i: (0, i, 0)),
            pl.BlockSpec((n_blk, 1), lambda i: (i, 0)),
            pl.BlockSpec((n_blk, 1), lambda i: (i, 0)),
            pl.BlockSpec((d_out,), lambda i: (0,)),
        ],
        out_specs=pl.BlockSpec((n_blk, d_out), lambda i: (i, 0)),
        out_shape=jax.ShapeDtypeStruct((n, d_out), jnp.float32),
    )(outp, deg_a, deg_b, bias)


@jax.jit
def kernel(edge_index, pseudo, features, weight, bias):
    n, d_in = features.shape
    kk, _, d_out = weight.shape

    w2 = jnp.transpose(weight, (1, 0, 2)).reshape(d_in, kk * d_out)
    xw = _compute_xw(features, w2, n_blk=1000)

    src = edge_index[0]
    dst = edge_index[1]
    p0 = pseudo[:, 0]
    p1 = pseudo[:, 1]
    outp, degp = _sc_scatter(src, dst, p0, p1, xw, n, d_out)

    deg_a = degp[:n][:, None]
    deg_b = degp[n:][:, None]
    return _finalize(outp, deg_a, deg_b, bias, n_blk=1000)
